# Initial kernel scaffold; baseline (speedup 1.0000x reference)
import jax
import jax.numpy as jnp
from jax.experimental import pallas as pl


def kernel(x):
    raise NotImplementedError("write your pallas kernel here")


# speedup vs baseline: 1.9840x; 1.9840x over previous
import jax
import jax.numpy as jnp
from jax import lax
from jax.experimental import pallas as pl

_K = 32
_BITS_TWO = 0x40000000
_BITS_INF = 0x7F800000
_N = 32768
_ROWS, _COLS = 256, 128


def _body(x_ref, y_ref):
    x = x_ref[...]
    u = lax.bitcast_convert_type(jnp.maximum(x, 2.0), jnp.int32)

    def val_step(_, lohi):
        lo, hi = lohi
        mid = lo + (hi - lo) // 2
        c = jnp.sum((u >= mid).astype(jnp.int32))
        big = c >= _K
        return jnp.where(big, mid, lo), jnp.where(big, hi, mid)

    lo, _ = lax.fori_loop(0, 30, val_step, (jnp.int32(_BITS_TWO), jnp.int32(_BITS_INF)))
    t = lo

    c_gt = jnp.sum((u > t).astype(jnp.int32))
    m = jnp.where(t == _BITS_TWO, 0, _K - c_gt)

    idx = lax.broadcasted_iota(jnp.int32, (_ROWS, _COLS), 0) * _COLS + \
        lax.broadcasted_iota(jnp.int32, (_ROWS, _COLS), 1)
    tie = u == t

    def idx_step(_, lohi):
        lo2, hi2 = lohi
        mid = lo2 + (hi2 - lo2) // 2
        c = jnp.sum((tie & (idx < mid)).astype(jnp.int32))
        small = c < m
        return jnp.where(small, mid, lo2), jnp.where(small, hi2, mid)

    _, cut = lax.fori_loop(0, 15, idx_step, (jnp.int32(0), jnp.int32(_N)))

    win = (u > t) | (tie & (idx < cut) & (m > 0))
    y_ref[...] = jnp.where(win, 1.0, 0.0).astype(jnp.float32)


def kernel(x):
    x2 = x.reshape(_ROWS, _COLS)
    y = pl.pallas_call(
        _body,
        out_shape=jax.ShapeDtypeStruct((_ROWS, _COLS), jnp.float32),
    )(x2)
    return y.reshape(_N)
